# baseline (speedup 1.0000x reference)
import functools

import jax
import jax.numpy as jnp
from jax import lax
from jax.experimental import pallas as pl
from jax.experimental.pallas import tpu as pltpu
from jax.experimental.pallas import tpu_sc as plsc

B, S, H = 1, 2048, 1024
N, K = 8, 2
R = 2
V = 32000
I = 128
SCALE = 2.0

BT = 256
BN = 1280

NW = 32
TW = S // NW
L = 16

_EPS = jnp.finfo(jnp.float32).eps
_NEG = jnp.float32(-jnp.inf)


def _dot_t(a, b):
    return jax.lax.dot_general(a, b, (((1,), (1,)), ((), ())),
                               preferred_element_type=jnp.float32)


def _dot(a, b):
    return jax.lax.dot_general(a, b, (((1,), (0,)), ((), ())),
                               preferred_element_type=jnp.float32)


def _a_kernel(x_ref, Acat_ref, Bcat_ref, oA_ref, oB_ref, n1w_ref, Wr_ref,
              x1_ref, lg_ref):
    xa = x_ref[...]
    t = _dot_t(xa, Acat_ref[...])
    s = jax.lax.dot_general(
        t, Bcat_ref[...], (((1,), (1,)), ((), ())),
        preferred_element_type=jnp.float32) * SCALE
    t2 = _dot_t(s, oA_ref[...])
    a = jax.lax.dot_general(t2, oB_ref[...], (((1,), (1,)), ((), ())),
                            preferred_element_type=jnp.float32) * SCALE
    h1 = xa + a
    ms1 = jnp.mean(h1 * h1, axis=-1, keepdims=True)
    x1 = h1 * jax.lax.rsqrt(ms1 + _EPS) * n1w_ref[...]
    x1_ref[...] = x1
    lg = jax.lax.dot_general(Wr_ref[...], x1, (((1,), (1,)), ((), ())),
                             preferred_element_type=jnp.float32)
    lg_ref[...] = jnp.transpose(lg.reshape(N, BT // TW, TW), (1, 0, 2))


def _route_sc(lg_hbm, w_hbm, lbuf, wbuf):
    wid = lax.axis_index("s") * 2 + lax.axis_index("c")
    pltpu.sync_copy(lg_hbm.at[wid], lbuf)
    for c in range(TW // L):
        ls = [lbuf[e, pl.ds(c * L, L)] for e in range(N)]
        m1 = ls[0]
        for e in range(1, N):
            m1 = jnp.maximum(m1, ls[e])
        e1 = jnp.full((L,), N, jnp.int32)
        for e in range(N - 1, -1, -1):
            e1 = jnp.where(ls[e] == m1, jnp.full((L,), e, jnp.int32), e1)
        m2 = jnp.full((L,), _NEG, jnp.float32)
        for e in range(N):
            le = jnp.where(e1 == e, _NEG, ls[e])
            m2 = jnp.maximum(m2, le)
        e2 = jnp.full((L,), N, jnp.int32)
        for e in range(N - 1, -1, -1):
            hit = jnp.logical_and(ls[e] == m2, e1 != e)
            e2 = jnp.where(hit, jnp.full((L,), e, jnp.int32), e2)
        p = jnp.exp(m2 - m1)
        w1 = 1.0 / (1.0 + p)
        w2 = p * w1
        for e in range(N):
            we = jnp.where(e1 == e, w1, jnp.where(e2 == e, w2,
                                                  jnp.zeros((L,), jnp.float32)))
            wbuf[e, pl.ds(c * L, L)] = we
    pltpu.sync_copy(wbuf, w_hbm.at[wid])


def _b1_kernel(x1_ref, gcat_ref, gbf_ref, act_ref):
    x1b = x1_ref[...].astype(jnp.bfloat16)
    g = _dot_t(x1b, gcat_ref[...]) + gbf_ref[...]
    act = g * jax.nn.sigmoid(g)
    act_ref[...] = act.astype(jnp.bfloat16)


def _c_kernel(x1_ref, act_ref, wlg_ref, ucat_ref, ub_ref, n2w_ref,
              hw_ref, out_ref, x2_s):
    j = pl.program_id(0)

    @pl.when(j == 0)
    def _moe():
        def body(t, _):
            x1 = x1_ref[pl.ds(t * BT, BT), :]
            wl = wlg_ref[pl.ds(t * (BT // TW), BT // TW)]
            w = jnp.transpose(wl, (0, 2, 1)).reshape(BT, N)
            act = act_ref[pl.ds(t * BT, BT), :].astype(jnp.float32)
            w_exp = jnp.repeat(w, I, axis=1)
            actw = (act * w_exp).astype(jnp.bfloat16)
            acc = _dot(actw, ucat_ref[...]) + _dot(w, ub_ref[...])
            h2 = x1 + acc
            ms2 = jnp.mean(h2 * h2, axis=-1, keepdims=True)
            x2_s[pl.ds(t * BT, BT), :] = (
                h2 * jax.lax.rsqrt(ms2 + _EPS) * n2w_ref[...])
            return _

        jax.lax.fori_loop(0, S // BT, body, 0)

    out_ref[...] = _dot_t(x2_s[...], hw_ref[...])


def _full(shape):
    nd = len(shape)
    return pl.BlockSpec(shape, lambda i: (0,) * nd)


def kernel(x, qA, qB, kA, kB, vA, vB, oA, oB, n1w, n2w, Wr, gW, gb, uW, ub, headW):
    xf = x.reshape(S, H)
    Acat = jnp.concatenate([qA, kA, vA], axis=0)
    Bcat = jnp.concatenate([qB, kB, vB], axis=1)
    n1w2 = n1w.reshape(1, H)
    n2w2 = n2w.reshape(1, H)
    gcat = gW.reshape(N * I, H).astype(jnp.bfloat16)
    gbf = gb.reshape(1, N * I)
    ucat = jnp.transpose(uW, (0, 2, 1)).reshape(N * I, H).astype(jnp.bfloat16)

    x1, lg = pl.pallas_call(
        _a_kernel,
        grid=(S // BT,),
        in_specs=[
            pl.BlockSpec((BT, H), lambda i: (i, 0)),
            _full((6, H)), _full((H, 6)), _full((R, H)), _full((H, R)),
            _full((1, H)), _full((N, H)),
        ],
        out_specs=[
            pl.BlockSpec((BT, H), lambda i: (i, 0)),
            pl.BlockSpec((BT // TW, N, TW), lambda i: (i, 0, 0)),
        ],
        out_shape=[
            jax.ShapeDtypeStruct((S, H), jnp.float32),
            jax.ShapeDtypeStruct((NW, N, TW), jnp.float32),
        ],
    )(xf, Acat, Bcat, oA, oB, n1w2, Wr)

    route = functools.partial(
        pl.kernel,
        mesh=plsc.VectorSubcoreMesh(core_axis_name="c", subcore_axis_name="s"),
        out_type=jax.ShapeDtypeStruct((NW, N, TW), jnp.float32),
        scratch_types=[
            pltpu.VMEM((N, TW), jnp.float32),
            pltpu.VMEM((N, TW), jnp.float32),
        ],
    )(_route_sc)
    wlg = route(lg)

    act = pl.pallas_call(
        _b1_kernel,
        grid=(S // BT,),
        in_specs=[
            pl.BlockSpec((BT, H), lambda i: (i, 0)),
            _full((N * I, H)), _full((1, N * I)),
        ],
        out_specs=pl.BlockSpec((BT, N * I), lambda i: (i, 0)),
        out_shape=jax.ShapeDtypeStruct((S, N * I), jnp.bfloat16),
    )(x1, gcat, gbf)

    out = pl.pallas_call(
        _c_kernel,
        grid=(V // BN,),
        in_specs=[
            pl.BlockSpec((S, H), lambda j: (0, 0)),
            pl.BlockSpec((S, N * I), lambda j: (0, 0)),
            pl.BlockSpec((NW, N, TW), lambda j: (0, 0, 0)),
            _full((N * I, H)), _full((N, H)),
            _full((1, H)),
            pl.BlockSpec((BN, H), lambda j: (j, 0)),
        ],
        out_specs=pl.BlockSpec((S, BN), lambda j: (0, j)),
        out_shape=jax.ShapeDtypeStruct((S, V), jnp.float32),
        scratch_shapes=[pltpu.VMEM((S, H), jnp.float32)],
    )(x1, act, wlg, ucat, ub, n2w2, headW)

    return out.reshape(B, S, V)


# speedup vs baseline: 1.0621x; 1.0621x over previous
import functools

import jax
import jax.numpy as jnp
from jax import lax
from jax.experimental import pallas as pl
from jax.experimental.pallas import tpu as pltpu
from jax.experimental.pallas import tpu_sc as plsc

B, S, H = 1, 2048, 1024
N, K = 8, 2
R = 2
V = 32000
I = 128
SCALE = 2.0

BT = 512
BN = 1280

NW = 32
TW = S // NW
L = 16

_EPS = jnp.finfo(jnp.float32).eps
_NEG = jnp.float32(-jnp.inf)


def _dot_t(a, b):
    return jax.lax.dot_general(a, b, (((1,), (1,)), ((), ())),
                               preferred_element_type=jnp.float32)


def _dot(a, b):
    return jax.lax.dot_general(a, b, (((1,), (0,)), ((), ())),
                               preferred_element_type=jnp.float32)


def _a_kernel(x_ref, Acat_ref, Bcat_ref, oA_ref, oB_ref, n1w_ref, Wr_ref,
              x1_ref, lg_ref):
    xa = x_ref[...]
    t = _dot_t(xa, Acat_ref[...])
    s = jax.lax.dot_general(
        t, Bcat_ref[...], (((1,), (1,)), ((), ())),
        preferred_element_type=jnp.float32) * SCALE
    t2 = _dot_t(s, oA_ref[...])
    a = jax.lax.dot_general(t2, oB_ref[...], (((1,), (1,)), ((), ())),
                            preferred_element_type=jnp.float32) * SCALE
    h1 = xa + a
    ms1 = jnp.mean(h1 * h1, axis=-1, keepdims=True)
    x1 = h1 * jax.lax.rsqrt(ms1 + _EPS) * n1w_ref[...]
    x1_ref[...] = x1
    lg = jax.lax.dot_general(Wr_ref[...], x1, (((1,), (1,)), ((), ())),
                             preferred_element_type=jnp.float32)
    lg_ref[...] = jnp.transpose(lg.reshape(N, BT // TW, TW), (1, 0, 2))


def _route_sc(lg_hbm, w_hbm, lbuf, wbuf):
    wid = lax.axis_index("s") * 2 + lax.axis_index("c")
    pltpu.sync_copy(lg_hbm.at[wid], lbuf)
    for c in range(TW // L):
        ls = [lbuf[e, pl.ds(c * L, L)] for e in range(N)]
        m1 = ls[0]
        for e in range(1, N):
            m1 = jnp.maximum(m1, ls[e])
        e1 = jnp.full((L,), N, jnp.int32)
        for e in range(N - 1, -1, -1):
            e1 = jnp.where(ls[e] == m1, jnp.full((L,), e, jnp.int32), e1)
        m2 = jnp.full((L,), _NEG, jnp.float32)
        for e in range(N):
            le = jnp.where(e1 == e, _NEG, ls[e])
            m2 = jnp.maximum(m2, le)
        e2 = jnp.full((L,), N, jnp.int32)
        for e in range(N - 1, -1, -1):
            hit = jnp.logical_and(ls[e] == m2, e1 != e)
            e2 = jnp.where(hit, jnp.full((L,), e, jnp.int32), e2)
        p = jnp.exp(m2 - m1)
        w1 = 1.0 / (1.0 + p)
        w2 = p * w1
        for e in range(N):
            we = jnp.where(e1 == e, w1, jnp.where(e2 == e, w2,
                                                  jnp.zeros((L,), jnp.float32)))
            wbuf[e, pl.ds(c * L, L)] = we
    pltpu.sync_copy(wbuf, w_hbm.at[wid])


def _c_kernel(x1_ref, wlg_ref, gcat_ref, gbf_ref, ucat_ref, ub_ref, n2w_ref,
              hw_ref, out_ref, x2_s):
    j = pl.program_id(0)

    @pl.when(j == 0)
    def _moe():
        def body(t, _):
            x1 = x1_ref[pl.ds(t * BT, BT), :]
            wl = wlg_ref[pl.ds(t * (BT // TW), BT // TW)]
            w = jnp.transpose(wl, (0, 2, 1)).reshape(BT, N)
            x1b = x1.astype(jnp.bfloat16)
            g = _dot_t(x1b, gcat_ref[...]) + gbf_ref[...]
            act = g * jax.nn.sigmoid(g)
            w_exp = jnp.repeat(w, I, axis=1)
            actw = (act * w_exp).astype(jnp.bfloat16)
            acc = _dot(actw, ucat_ref[...]) + _dot(w, ub_ref[...])
            h2 = x1 + acc
            ms2 = jnp.mean(h2 * h2, axis=-1, keepdims=True)
            x2_s[pl.ds(t * BT, BT), :] = (
                h2 * jax.lax.rsqrt(ms2 + _EPS) * n2w_ref[...])
            return _

        jax.lax.fori_loop(0, S // BT, body, 0)

    out_ref[...] = _dot_t(x2_s[...], hw_ref[...])


def _full(shape):
    nd = len(shape)
    return pl.BlockSpec(shape, lambda i: (0,) * nd)


def kernel(x, qA, qB, kA, kB, vA, vB, oA, oB, n1w, n2w, Wr, gW, gb, uW, ub, headW):
    xf = x.reshape(S, H)
    Acat = jnp.concatenate([qA, kA, vA], axis=0)
    Bcat = jnp.concatenate([qB, kB, vB], axis=1)
    n1w2 = n1w.reshape(1, H)
    n2w2 = n2w.reshape(1, H)
    gcat = gW.reshape(N * I, H).astype(jnp.bfloat16)
    gbf = gb.reshape(1, N * I)
    ucat = jnp.transpose(uW, (0, 2, 1)).reshape(N * I, H).astype(jnp.bfloat16)

    x1, lg = pl.pallas_call(
        _a_kernel,
        grid=(S // BT,),
        in_specs=[
            pl.BlockSpec((BT, H), lambda i: (i, 0)),
            _full((6, H)), _full((H, 6)), _full((R, H)), _full((H, R)),
            _full((1, H)), _full((N, H)),
        ],
        out_specs=[
            pl.BlockSpec((BT, H), lambda i: (i, 0)),
            pl.BlockSpec((BT // TW, N, TW), lambda i: (i, 0, 0)),
        ],
        out_shape=[
            jax.ShapeDtypeStruct((S, H), jnp.float32),
            jax.ShapeDtypeStruct((NW, N, TW), jnp.float32),
        ],
    )(xf, Acat, Bcat, oA, oB, n1w2, Wr)

    route = functools.partial(
        pl.kernel,
        mesh=plsc.VectorSubcoreMesh(core_axis_name="c", subcore_axis_name="s"),
        out_type=jax.ShapeDtypeStruct((NW, N, TW), jnp.float32),
        scratch_types=[
            pltpu.VMEM((N, TW), jnp.float32),
            pltpu.VMEM((N, TW), jnp.float32),
        ],
    )(_route_sc)
    wlg = route(lg)

    out = pl.pallas_call(
        _c_kernel,
        grid=(V // BN,),
        in_specs=[
            pl.BlockSpec((S, H), lambda j: (0, 0)),
            pl.BlockSpec((NW, N, TW), lambda j: (0, 0, 0)),
            _full((N * I, H)), _full((1, N * I)), _full((N * I, H)), _full((N, H)),
            _full((1, H)),
            pl.BlockSpec((BN, H), lambda j: (j, 0)),
        ],
        out_specs=pl.BlockSpec((S, BN), lambda j: (0, j)),
        out_shape=jax.ShapeDtypeStruct((S, V), jnp.float32),
        scratch_shapes=[pltpu.VMEM((S, H), jnp.float32)],
    )(x1, wlg, gcat, gbf, ucat, ub, n2w2, headW)

    return out.reshape(B, S, V)
